# baseline (speedup 1.0000x reference)
import jax
import jax.numpy as jnp
from jax import lax
from jax.experimental import pallas as pl
from jax.experimental.pallas import tpu as pltpu
from jax.experimental.pallas import tpu_sc as plsc

B = 1024
SEQ = 200
EMB = 64
NC = 2
NS = 16
NW = NC * NS
BT = B // 128
NU = SEQ * BT
UPW = NU // NW
NBUF = 4
PRE = 2
LANES = 16
ET = EMB // 8
SHI = SEQ // 8


def _body(qi_hbm, ci_hbm, qtab_hbm, ctab_hbm, pos_hbm, out_hbm,
          qi, ci, pos_v, q_buf, c_buf, o_buf, gsem, osem):
    wid = lax.axis_index("s") * NC + lax.axis_index("c")
    u0 = wid * UPW
    s_first = u0 // BT
    row0 = jnp.minimum(s_first, SEQ - 8)
    pltpu.sync_copy(pos_hbm.at[pl.ds(row0, 8)], pos_v)
    idx_copies = []
    for t in range(8):
        s = row0 + t
        shi, slo = s // 8, lax.rem(s, 8)
        idx_copies.append(pltpu.async_copy(qi_hbm.at[shi, :, slo], qi.at[t],
                                           gsem[0]))
        idx_copies.append(pltpu.async_copy(ci_hbm.at[shi, :, slo], ci.at[t],
                                           gsem[0]))
    for c in idx_copies:
        c.wait()

    def _unit(cur):
        u = u0 + cur
        s = u // BT
        bt = lax.rem(u, BT)
        return s, bt, s - row0

    def _gather_refs(cur, slot):
        _, bt, lr = _unit(cur)
        return (
            (qtab_hbm.at[qi.at[lr, bt]], q_buf.at[slot]),
            (ctab_hbm.at[ci.at[lr, bt]], c_buf.at[slot]),
        )

    def fire(cur, slot):
        for src, dst in _gather_refs(cur, slot):
            pltpu.async_copy(src, dst, gsem[slot])

    def wait_gathers(cur, slot):
        for src, dst in _gather_refs(cur, slot):
            pltpu.make_async_copy(src, dst, gsem[slot]).wait()

    def compute(cur, slot):
        _, _, lr = _unit(cur)
        iota = lax.iota(jnp.int32, LANES)
        psg = [pos_v[lr, pl.ds(g * LANES, LANES)] for g in range(EMB // LANES)]
        eidx = [((g * LANES + iota) // 8, (g * LANES + iota) % 8)
                for g in range(EMB // LANES)]

        @plsc.parallel_loop(0, 128, unroll=8)
        def _item(i):
            bb = jnp.full((LANES,), i, jnp.int32)
            for g in range(EMB // LANES):
                qv = q_buf[slot, i, pl.ds(g * LANES, LANES)]
                cv = c_buf[slot, i, pl.ds(g * LANES, LANES)]
                plsc.store_scatter(o_buf.at[slot], [eidx[g][0], eidx[g][1], bb],
                                   qv + cv + psg[g])

    def fire_write(cur, slot):
        s, bt, _ = _unit(cur)
        pltpu.async_copy(o_buf.at[slot, :, :, pl.ds(0, 128)],
                         out_hbm.at[s, :, bt], osem[slot])

    def wait_write(cur, slot):
        s, bt, _ = _unit(cur)
        pltpu.make_async_copy(o_buf.at[slot, :, :, pl.ds(0, 128)],
                              out_hbm.at[s, :, bt], osem[slot]).wait()

    for j in range(PRE):
        fire(j, j % NBUF)

    LOOPED = (UPW // NBUF) * NBUF

    @pl.loop(0, LOOPED, step=NBUF)
    def _turns(k):
        for b in range(NBUF):
            cur = k + b
            wait_gathers(cur, b)
            compute(cur, b)
            fire_write(cur, b)
            nxt = cur + PRE
            bn = (b + PRE) % NBUF

            @pl.when(nxt < UPW)
            def _():
                @pl.when(cur >= NBUF - PRE)
                def _():
                    wait_write(nxt - NBUF, bn)
                fire(nxt, bn)

    for cur in range(LOOPED, UPW):
        b = cur % NBUF
        wait_gathers(cur, b)
        compute(cur, b)
        fire_write(cur, b)

    for cur in range(UPW - NBUF, UPW):
        wait_write(cur, cur % NBUF)


def kernel(questions, category, question_table, category_table, pos_table):
    def native_idx(x):
        return (x.astype(jnp.int32).reshape(BT, 128, SHI, 8)
                .transpose(2, 0, 3, 1))

    out5d = pl.kernel(
        _body,
        out_type=jax.ShapeDtypeStruct((SEQ, ET, BT, 8, 128), jnp.float32),
        mesh=plsc.VectorSubcoreMesh(core_axis_name="c", subcore_axis_name="s"),
        compiler_params=pltpu.CompilerParams(use_tc_tiling_on_sc=False,
                                             needs_layout_passes=False),
        scratch_types=[
            pltpu.VMEM((8, BT, 128), jnp.int32),
            pltpu.VMEM((8, BT, 128), jnp.int32),
            pltpu.VMEM((8, EMB), jnp.float32),
            pltpu.VMEM((NBUF, 128, EMB), jnp.float32),
            pltpu.VMEM((NBUF, 128, EMB), jnp.float32),
            pltpu.VMEM((NBUF, ET, 8, 129), jnp.float32),
            [pltpu.SemaphoreType.DMA] * NBUF,
            [pltpu.SemaphoreType.DMA] * NBUF,
        ],
    )(native_idx(questions), native_idx(category),
      question_table, category_table, pos_table)
    return out5d.transpose(2, 4, 0, 1, 3).reshape(B, SEQ, EMB)


# speedup vs baseline: 1.0122x; 1.0122x over previous
import jax
import jax.numpy as jnp
from jax import lax
from jax.experimental import pallas as pl
from jax.experimental.pallas import tpu as pltpu
from jax.experimental.pallas import tpu_sc as plsc

B = 1024
SEQ = 200
EMB = 64
NC = 2
NS = 16
NW = NC * NS
BT = B // 128
NU = SEQ * BT
UPW = NU // NW
NBUF = 4
PRE = 2
LANES = 16
ET = EMB // 8
SHI = SEQ // 8


def _body(qi_hbm, ci_hbm, qtab_hbm, ctab_hbm, pos_hbm, out_hbm,
          qi, ci, pos_v, q_buf, c_buf, o_buf, gsem, osem):
    wid = lax.axis_index("s") * NC + lax.axis_index("c")
    u0 = wid * UPW
    s_first = u0 // BT
    row0 = jnp.minimum(s_first, SEQ - 8)
    pltpu.sync_copy(pos_hbm.at[pl.ds(row0, 8)], pos_v)
    idx_copies = []
    for t in range(8):
        s = row0 + t
        shi, slo = s // 8, lax.rem(s, 8)
        idx_copies.append(pltpu.async_copy(qi_hbm.at[shi, :, slo], qi.at[t],
                                           gsem[0]))
        idx_copies.append(pltpu.async_copy(ci_hbm.at[shi, :, slo], ci.at[t],
                                           gsem[0]))
    for c in idx_copies:
        c.wait()

    def _unit(cur):
        u = u0 + cur
        s = u // BT
        bt = lax.rem(u, BT)
        return s, bt, s - row0

    def _gather_refs(cur, slot):
        _, bt, lr = _unit(cur)
        return (
            (qtab_hbm.at[qi.at[lr, bt]], q_buf.at[slot]),
            (ctab_hbm.at[ci.at[lr, bt]], c_buf.at[slot]),
        )

    def fire(cur, slot):
        for src, dst in _gather_refs(cur, slot):
            pltpu.async_copy(src, dst, gsem[slot])

    def wait_gathers(cur, slot):
        for src, dst in _gather_refs(cur, slot):
            pltpu.make_async_copy(src, dst, gsem[slot]).wait()

    def compute(cur, slot):
        _, _, lr = _unit(cur)
        iota = lax.iota(jnp.int32, LANES)
        psg = [pos_v[lr, pl.ds(g * LANES, LANES)] for g in range(EMB // LANES)]
        eidx = [((g * LANES + iota) // 8, (g * LANES + iota) % 8)
                for g in range(EMB // LANES)]

        @plsc.parallel_loop(0, 128, unroll=4)
        def _item(i):
            bb = jnp.full((LANES,), i, jnp.int32)
            for g in range(EMB // LANES):
                qv = q_buf[slot, i, pl.ds(g * LANES, LANES)]
                cv = c_buf[slot, i, pl.ds(g * LANES, LANES)]
                plsc.store_scatter(o_buf.at[slot], [eidx[g][0], eidx[g][1], bb],
                                   qv + cv + psg[g])

    def fire_write(cur, slot):
        s, bt, _ = _unit(cur)
        pltpu.async_copy(o_buf.at[slot, :, :, pl.ds(0, 128)],
                         out_hbm.at[s, :, bt], osem[slot])

    def wait_write(cur, slot):
        s, bt, _ = _unit(cur)
        pltpu.make_async_copy(o_buf.at[slot, :, :, pl.ds(0, 128)],
                              out_hbm.at[s, :, bt], osem[slot]).wait()

    for j in range(PRE):
        fire(j, j % NBUF)

    LOOPED = (UPW // NBUF) * NBUF

    @pl.loop(0, LOOPED, step=NBUF)
    def _turns(k):
        for b in range(NBUF):
            cur = k + b
            wait_gathers(cur, b)
            compute(cur, b)
            fire_write(cur, b)
            nxt = cur + PRE
            bn = (b + PRE) % NBUF

            @pl.when(nxt < UPW)
            def _():
                @pl.when(cur >= NBUF - PRE)
                def _():
                    wait_write(nxt - NBUF, bn)
                fire(nxt, bn)

    for cur in range(LOOPED, UPW):
        b = cur % NBUF
        wait_gathers(cur, b)
        compute(cur, b)
        fire_write(cur, b)

    for cur in range(UPW - NBUF, UPW):
        wait_write(cur, cur % NBUF)


def kernel(questions, category, question_table, category_table, pos_table):
    def native_idx(x):
        return (x.astype(jnp.int32).reshape(BT, 128, SHI, 8)
                .transpose(2, 0, 3, 1))

    out5d = pl.kernel(
        _body,
        out_type=jax.ShapeDtypeStruct((SEQ, ET, BT, 8, 128), jnp.float32),
        mesh=plsc.VectorSubcoreMesh(core_axis_name="c", subcore_axis_name="s"),
        compiler_params=pltpu.CompilerParams(use_tc_tiling_on_sc=False,
                                             needs_layout_passes=False),
        scratch_types=[
            pltpu.VMEM((8, BT, 128), jnp.int32),
            pltpu.VMEM((8, BT, 128), jnp.int32),
            pltpu.VMEM((8, EMB), jnp.float32),
            pltpu.VMEM((NBUF, 128, EMB), jnp.float32),
            pltpu.VMEM((NBUF, 128, EMB), jnp.float32),
            pltpu.VMEM((NBUF, ET, 8, 129), jnp.float32),
            [pltpu.SemaphoreType.DMA] * NBUF,
            [pltpu.SemaphoreType.DMA] * NBUF,
        ],
    )(native_idx(questions), native_idx(category),
      question_table, category_table, pos_table)
    return out5d.transpose(2, 4, 0, 1, 3).reshape(B, SEQ, EMB)
